# baseline (speedup 1.0000x reference)
import jax
import jax.numpy as jnp
from jax import lax
from jax.experimental import pallas as pl
from jax.experimental.pallas import tpu as pltpu
from jax.experimental.pallas import tpu_sc as plsc

_NC = 2
_NS = 16
_NW = _NC * _NS
_NH = 4


@jax.jit
def _sc_emb(tab_t, x_t):
    r_total, v = tab_t.shape
    f, b = x_t.shape
    d = r_total // f
    chunk = b // _NH
    w_len = ((v // 2) + 127) // 128 * 128
    npair = f // 2

    mesh = plsc.VectorSubcoreMesh(
        core_axis_name="c", subcore_axis_name="s",
        num_cores=_NC, num_subcores=_NS)

    def body(tab_hbm, x_hbm, out_hbm, tw_v, idx_v, orow_v, xshs,
             twsems, xsems, isems, osems):
        cid = lax.axis_index("c")
        sid = lax.axis_index("s")
        wid = sid * _NC + cid
        lane = jnp.arange(16, dtype=jnp.int32)

        def tw_copy(j, w):
            off = w * w_len
            return pltpu.make_async_copy(
                tab_hbm.at[d * j + wid].at[
                    pl.ds(off, (v - w_len) if w else w_len)],
                tw_v[w], twsems[w])

        def idx_copy(par, hh):
            h = hh % _NH
            return pltpu.make_async_copy(
                xshs[par].at[pl.ds(h * chunk, chunk)], idx_v[hh % 2],
                isems[hh % 2])

        def st_copy(j, h):
            return pltpu.make_async_copy(
                orow_v.at[pl.ds(h * chunk, chunk)],
                out_hbm.at[d * j + wid, pl.ds(h * chunk, chunk)],
                osems[h])

        def xsh_copy(j, par):
            return pltpu.make_async_copy(
                x_hbm.at[j], xshs[par], xsems[par])

        @pl.when(sid == 0)
        def _():
            xsh_copy(0, 0).start()
            xsh_copy(1, 1).start()
            xsh_copy(0, 0).wait()

        plsc.subcore_barrier()
        idx_copy(0, 0).start()
        tw_copy(0, 0).start()
        tw_copy(0, 1).start()

        def loop_body(k, _):
            for par in range(2):
                j = 2 * k + par
                tw_copy(j, 0).wait()
                for h in range(_NH):
                    idx_copy(par, h).wait()
                    idx_copy(par, h + 1).start()
                    if par == 0:
                        @pl.when(j > 0)
                        def _():
                            st_copy(j - 1, h).wait()
                    else:
                        st_copy(j - 1, h).wait()
                    base = h * chunk

                    @plsc.parallel_loop(0, chunk // 16, 1, unroll=8)
                    def pass0(t):
                        iv = idx_v[h % 2][pl.ds(t * 16, 16)]
                        orow_v[pl.ds(base + t * 16, 16)] = (
                            plsc.load_gather(tw_v[0], [iv],
                                             mask=iv < w_len))

                if par == 0:
                    tw_copy(j + 1, 0).start()
                else:
                    @pl.when(k + 1 < npair)
                    def _():
                        tw_copy(j + 1, 0).start()
                tw_copy(j, 1).wait()
                for h in range(_NH):
                    hh = _NH + h
                    idx_copy(par, hh).wait()
                    if hh + 1 < 2 * _NH:
                        idx_copy(par, hh + 1).start()
                    else:
                        last = (par == 1) and (k + 1 >= npair)
                        if par == 0:
                            @pl.when(sid == 0)
                            def _():
                                xsh_copy(j + 1, 1).wait()

                            plsc.subcore_barrier()

                            @pl.when((sid == 0) & (k + 1 < npair))
                            def _():
                                xsh_copy(j + 2, 0).start()

                            idx_copy(1, 0).start()
                        else:
                            @pl.when(k + 1 < npair)
                            def _():
                                @pl.when(sid == 0)
                                def _():
                                    xsh_copy(j + 1, 0).wait()

                                plsc.subcore_barrier()

                                @pl.when(sid == 0)
                                def _():
                                    xsh_copy(j + 2, 1).start()

                                idx_copy(0, 0).start()
                    base = h * chunk

                    @plsc.parallel_loop(0, chunk // 16, 1, unroll=8)
                    def pass1(t):
                        iv = idx_v[hh % 2][pl.ds(t * 16, 16)]
                        m1 = iv >= w_len
                        g1 = plsc.load_gather(tw_v[1], [iv - w_len],
                                              mask=m1)
                        plsc.store_scatter(
                            orow_v, [base + t * 16 + lane], g1, mask=m1)

                    st_copy(j, h).start()
                if par == 0:
                    tw_copy(j + 1, 1).start()
                else:
                    @pl.when(k + 1 < npair)
                    def _():
                        tw_copy(j + 1, 1).start()
            return 0

        lax.fori_loop(0, npair, loop_body, 0)
        for h in range(_NH):
            st_copy(f - 1, h).wait()

    return pl.kernel(
        body,
        out_type=jax.ShapeDtypeStruct((r_total, b), jnp.float32),
        mesh=mesh,
        scratch_types=[
            [pltpu.VMEM((w_len,), jnp.float32),
             pltpu.VMEM((v - w_len,), jnp.float32)],
            [pltpu.VMEM((chunk,), jnp.int32) for _ in range(2)],
            pltpu.VMEM((b,), jnp.float32),
            [pltpu.VMEM_SHARED((b,), jnp.int32) for _ in range(2)],
            [pltpu.SemaphoreType.DMA for _ in range(2)],
            [pltpu.SemaphoreType.DMA for _ in range(2)],
            [pltpu.SemaphoreType.DMA for _ in range(2)],
            [pltpu.SemaphoreType.DMA for _ in range(_NH)],
        ],
        compiler_params=pltpu.CompilerParams(
            use_tc_tiling_on_sc=True, needs_layout_passes=False),
    )(tab_t, x_t)


def kernel(x, tables):
    f, v, d = tables.shape
    tab_t = jnp.swapaxes(tables, 1, 2).reshape(f * d, v)
    x_t = x.T.astype(jnp.int32)
    out_t = _sc_emb(tab_t, x_t)
    return out_t.T


# speedup vs baseline: 1.0051x; 1.0051x over previous
import jax
import jax.numpy as jnp
from jax import lax
from jax.experimental import pallas as pl
from jax.experimental.pallas import tpu as pltpu
from jax.experimental.pallas import tpu_sc as plsc

_NC = 2
_NS = 16
_NW = _NC * _NS
_NH = 4


@jax.jit
def _sc_emb(tab_t, x_t):
    r_total, v = tab_t.shape
    f, b = x_t.shape
    d = r_total // f
    chunk = b // _NH
    w_len = ((v // 2) + 127) // 128 * 128
    npair = f // 2

    mesh = plsc.VectorSubcoreMesh(
        core_axis_name="c", subcore_axis_name="s",
        num_cores=_NC, num_subcores=_NS)

    def body(tab_hbm, x_hbm, out_hbm, tw_v, idx_v, orow_v, xshs,
             twsems, xsems, isems, osems):
        cid = lax.axis_index("c")
        sid = lax.axis_index("s")
        wid = sid * _NC + cid
        lane = jnp.arange(16, dtype=jnp.int32)

        def tw_copy(j, w):
            off = w * w_len
            return pltpu.make_async_copy(
                tab_hbm.at[d * j + wid].at[
                    pl.ds(off, (v - w_len) if w else w_len)],
                tw_v[w], twsems[w])

        def idx_copy(par, hh):
            h = hh % _NH
            return pltpu.make_async_copy(
                xshs[par].at[pl.ds(h * chunk, chunk)], idx_v[hh % 2],
                isems[hh % 2])

        def st_copy(j, h):
            return pltpu.make_async_copy(
                orow_v.at[pl.ds(h * chunk, chunk)],
                out_hbm.at[d * j + wid, pl.ds(h * chunk, chunk)],
                osems[h])

        def xsh_copy(j, par):
            return pltpu.make_async_copy(
                x_hbm.at[j], xshs[par], xsems[par])

        @pl.when(sid == 0)
        def _():
            xsh_copy(0, 0).start()
            xsh_copy(1, 1).start()
            xsh_copy(0, 0).wait()

        plsc.subcore_barrier()
        idx_copy(0, 0).start()
        tw_copy(0, 0).start()
        tw_copy(0, 1).start()

        def loop_body(k, _):
            for par in range(2):
                j = 2 * k + par
                tw_copy(j, 0).wait()
                for h in range(_NH):
                    idx_copy(par, h).wait()
                    idx_copy(par, h + 1).start()
                    if par == 0:
                        @pl.when(j > 0)
                        def _():
                            st_copy(j - 1, h).wait()
                    else:
                        st_copy(j - 1, h).wait()
                    base = h * chunk

                    @plsc.parallel_loop(0, chunk // 16, 1, unroll=16)
                    def pass0(t):
                        iv = idx_v[h % 2][pl.ds(t * 16, 16)]
                        orow_v[pl.ds(base + t * 16, 16)] = (
                            plsc.load_gather(tw_v[0], [iv],
                                             mask=iv < w_len))

                if par == 0:
                    tw_copy(j + 1, 0).start()
                else:
                    @pl.when(k + 1 < npair)
                    def _():
                        tw_copy(j + 1, 0).start()
                tw_copy(j, 1).wait()
                for h in range(_NH):
                    hh = _NH + h
                    idx_copy(par, hh).wait()
                    if hh + 1 < 2 * _NH:
                        idx_copy(par, hh + 1).start()
                    else:
                        last = (par == 1) and (k + 1 >= npair)
                        if par == 0:
                            @pl.when(sid == 0)
                            def _():
                                xsh_copy(j + 1, 1).wait()

                            plsc.subcore_barrier()

                            @pl.when((sid == 0) & (k + 1 < npair))
                            def _():
                                xsh_copy(j + 2, 0).start()

                            idx_copy(1, 0).start()
                        else:
                            @pl.when(k + 1 < npair)
                            def _():
                                @pl.when(sid == 0)
                                def _():
                                    xsh_copy(j + 1, 0).wait()

                                plsc.subcore_barrier()

                                @pl.when(sid == 0)
                                def _():
                                    xsh_copy(j + 2, 1).start()

                                idx_copy(0, 0).start()
                    base = h * chunk

                    @plsc.parallel_loop(0, chunk // 16, 1, unroll=16)
                    def pass1(t):
                        iv = idx_v[hh % 2][pl.ds(t * 16, 16)]
                        m1 = iv >= w_len
                        g1 = plsc.load_gather(tw_v[1], [iv - w_len],
                                              mask=m1)
                        plsc.store_scatter(
                            orow_v, [base + t * 16 + lane], g1, mask=m1)

                    st_copy(j, h).start()
                if par == 0:
                    tw_copy(j + 1, 1).start()
                else:
                    @pl.when(k + 1 < npair)
                    def _():
                        tw_copy(j + 1, 1).start()
            return 0

        lax.fori_loop(0, npair, loop_body, 0)
        for h in range(_NH):
            st_copy(f - 1, h).wait()

    return pl.kernel(
        body,
        out_type=jax.ShapeDtypeStruct((r_total, b), jnp.float32),
        mesh=mesh,
        scratch_types=[
            [pltpu.VMEM((w_len,), jnp.float32),
             pltpu.VMEM((v - w_len,), jnp.float32)],
            [pltpu.VMEM((chunk,), jnp.int32) for _ in range(2)],
            pltpu.VMEM((b,), jnp.float32),
            [pltpu.VMEM_SHARED((b,), jnp.int32) for _ in range(2)],
            [pltpu.SemaphoreType.DMA for _ in range(2)],
            [pltpu.SemaphoreType.DMA for _ in range(2)],
            [pltpu.SemaphoreType.DMA for _ in range(2)],
            [pltpu.SemaphoreType.DMA for _ in range(_NH)],
        ],
        compiler_params=pltpu.CompilerParams(
            use_tc_tiling_on_sc=True, needs_layout_passes=False),
    )(tab_t, x_t)


def kernel(x, tables):
    f, v, d = tables.shape
    tab_t = jnp.swapaxes(tables, 1, 2).reshape(f * d, v)
    x_t = x.T.astype(jnp.int32)
    out_t = _sc_emb(tab_t, x_t)
    return out_t.T
